# baseline (speedup 1.0000x reference)
import functools

import jax
import jax.numpy as jnp
from jax import lax
from jax.experimental import pallas as pl
from jax.experimental.pallas import tpu as pltpu
from jax.experimental.pallas import tpu_sc as plsc

NU = 50000
N = 2 * NU
D = 32
EH = 800000
NC, NS = 2, 16
CHUNK = 256
BODY_CHUNKS = 5
BODY_EDGES = BODY_CHUNKS * CHUNK
NBODY = 40
KSUB = NBODY * BODY_EDGES
EH_PAD = NS * KSUB
PADH = EH_PAD - EH
NBODIES = NC * NS * NBODY
SENT = NU
ACC_ROWS = NU + 8
SLICE = NU // NS
BATCH = 4096
GB = BATCH // (NC * NS)

_mesh = plsc.VectorSubcoreMesh(core_axis_name="c", subcore_axis_name="s")
_sc_params = pltpu.CompilerParams(use_tc_tiling_on_sc=False)
_sc_params_nl = pltpu.CompilerParams(use_tc_tiling_on_sc=False,
                                     needs_layout_passes=False)


@functools.partial(
    pl.kernel,
    out_type=jax.ShapeDtypeStruct((N, D), jnp.float32),
    mesh=_mesh,
    compiler_params=_sc_params_nl,
    scratch_types=[
        pltpu.VMEM_SHARED((ACC_ROWS, D), jnp.float32),
        pltpu.VMEM((BODY_EDGES,), jnp.int32),
        pltpu.VMEM((BODY_EDGES,), jnp.int32),
        pltpu.VMEM((BODY_CHUNKS * 2, 128), jnp.int32),
        pltpu.VMEM((BODY_CHUNKS * 2, 128), jnp.int32),
        pltpu.VMEM((CHUNK, D), jnp.float32),
        pltpu.VMEM((CHUNK, D), jnp.float32),
        pltpu.VMEM((CHUNK, D), jnp.bfloat16),
        pltpu.VMEM((CHUNK, D), jnp.bfloat16),
        pltpu.SemaphoreType.DMA,
        pltpu.SemaphoreType.DMA,
        pltpu.SemaphoreType.DMA,
        pltpu.SemaphoreType.DMA,
        pltpu.SemaphoreType.DMA,
        pltpu.SemaphoreType.DMA,
    ],
)
def _spmm(x_hbm, col_hbm, row_hbm, zero_hbm, out_hbm,
          acc_sh, colv0, colv1, rowv0, rowv1, rows0, rows1, rowsb0, rowsb1,
          sI0, sI1, sG0, sG1, sS0, sS1):
    c = lax.axis_index("c")
    s = lax.axis_index("s")
    pltpu.sync_copy(zero_hbm, acc_sh.at[pl.ds(s * SLICE, SLICE)])
    plsc.subcore_barrier()
    ebase = c * EH_PAD + s * KSUB
    bbase = (c * NS + s) * NBODY
    colv, rowv = (colv0, colv1), (rowv0, rowv1)
    rows, sI = (rows0, rows1), (sI0, sI1)
    rowsb = (rowsb0, rowsb1)
    sG, sS = (sG0, sG1), (sS0, sS1)

    def idx_descs(b, buf):
        return (
            pltpu.make_async_copy(
                col_hbm.at[pl.ds(ebase + b * BODY_EDGES, BODY_EDGES)],
                colv[buf], sI[buf]),
            pltpu.make_async_copy(row_hbm.at[bbase + b], rowv[buf], sI[buf]),
        )

    for d in idx_descs(0, 0):
        d.start()

    @pl.loop(0, NBODY // 2)
    def _(t):
        for bb in range(2):
            P, Q = bb, 1 - bb
            b = 2 * t + bb
            for d in idx_descs(b, P):
                d.wait()
            bn = jnp.minimum(b + 1, NBODY - 1)
            for d in idx_descs(bn, Q):
                d.start()

            def gather(i, p):
                return pltpu.async_copy(
                    x_hbm.at[colv[P].at[pl.ds(i * CHUNK, CHUNK)]],
                    rowsb[p], sG[p])

            gd = [gather(0, 0), None]
            pend = [[], []]
            for i in range(BODY_CHUNKS):
                p = i % 2
                q = 1 - p
                gd[p].wait()
                if i < BODY_CHUNKS - 1:
                    gd[q] = gather(i + 1, q)
                for d in pend[p]:
                    d.wait()
                pend[p] = []

                @pl.loop(0, CHUNK, step=4)
                def _(r):
                    for u in range(4):
                        ab = rowsb[p][r + u, :]
                        lo, hi = plsc.unpack(
                            ab, format=plsc.PackFormat.INTERLEAVED)
                        rows[p][r + u, pl.ds(0, 16)] = lo
                        rows[p][r + u, pl.ds(16, 16)] = hi

                for g in range(2):
                    pend[p].append(pltpu.async_copy(
                        rows[p].at[pl.ds(g * 128, 128)],
                        acc_sh.at[rowv[P].at[i * 2 + g]], sS[p], add=True))
            for p in range(2):
                for d in pend[p]:
                    d.wait()

    for d in idx_descs(NBODY - 1, 0):
        d.wait()
    plsc.subcore_barrier()
    pltpu.sync_copy(acc_sh.at[pl.ds(s * SLICE, SLICE)],
                    out_hbm.at[pl.ds(c * NU + s * SLICE, SLICE)])


@functools.partial(
    pl.kernel,
    out_type=jax.ShapeDtypeStruct((N, D), jnp.float32),
    mesh=_mesh,
    compiler_params=_sc_params,
    scratch_types=[
        pltpu.VMEM_SHARED((ACC_ROWS, D), jnp.float32),
        pltpu.VMEM((BODY_CHUNKS * 2, 128), jnp.int32),
        pltpu.VMEM((BODY_CHUNKS * 2, 128), jnp.int32),
        pltpu.VMEM((128, D), jnp.float32),
        pltpu.SemaphoreType.DMA,
        pltpu.SemaphoreType.DMA,
        pltpu.SemaphoreType.DMA,
    ],
)
def _degrees(row_hbm, ones_hbm, zero_hbm, out_hbm,
             acc_sh, rowv0, rowv1, onesv, sI0, sI1, sS):
    c = lax.axis_index("c")
    s = lax.axis_index("s")
    pltpu.sync_copy(zero_hbm, acc_sh.at[pl.ds(s * SLICE, SLICE)])
    pltpu.sync_copy(ones_hbm, onesv)
    plsc.subcore_barrier()
    bbase = (c * NS + s) * NBODY
    rowv, sI = (rowv0, rowv1), (sI0, sI1)

    def idx_desc(b, buf):
        return pltpu.make_async_copy(row_hbm.at[bbase + b], rowv[buf], sI[buf])

    idx_desc(0, 0).start()

    @pl.loop(0, NBODY // 2)
    def _(t):
        for bb in range(2):
            P, Q = bb, 1 - bb
            b = 2 * t + bb
            idx_desc(b, P).wait()
            bn = jnp.minimum(b + 1, NBODY - 1)
            idx_desc(bn, Q).start()
            pend = []
            for g in range(BODY_CHUNKS * 2):
                pend.append(pltpu.async_copy(
                    onesv, acc_sh.at[rowv[P].at[g]], sS, add=True))
            for d in pend:
                d.wait()

    idx_desc(NBODY - 1, 0).wait()
    plsc.subcore_barrier()
    pltpu.sync_copy(acc_sh.at[pl.ds(s * SLICE, SLICE)],
                    out_hbm.at[pl.ds(c * NU + s * SLICE, SLICE)])


@functools.partial(
    pl.kernel,
    out_type=tuple(jax.ShapeDtypeStruct((BATCH, D), jnp.float32)
                   for _ in range(6)),
    mesh=_mesh,
    compiler_params=_sc_params,
    scratch_types=[
        pltpu.VMEM((GB,), jnp.int32),
        pltpu.VMEM((GB, D), jnp.float32),
        pltpu.SemaphoreType.DMA,
    ],
)
def _triplet_gather(mean_hbm, w_hbm, idx_hbm,
                    o0, o1, o2, o3, o4, o5, idxv, rowsv, sem):
    wid = lax.axis_index("s") * NC + lax.axis_index("c")
    for t, out in enumerate((o0, o1, o2, o3, o4, o5)):
        tab = mean_hbm if t < 3 else w_hbm
        pltpu.sync_copy(idx_hbm.at[pl.ds(t * BATCH + wid * GB, GB)], idxv)
        pltpu.async_copy(tab.at[idxv], rowsv, sem).wait()
        pltpu.sync_copy(rowsv, out.at[pl.ds(wid * GB, GB)])


def _tau_bf16(x):
    b = x.shape[0]
    return (x.reshape(b, 4, 2, 16).transpose(0, 1, 3, 2)
            .reshape(b, 128).astype(jnp.bfloat16))


def _prep_body(deg_ref, w_ref, dinv_ref, x0_ref):
    d = deg_ref[...]
    dv = jnp.where(d > 0.0, lax.rsqrt(jnp.maximum(d, 1.0)), 0.0)
    dinv_ref[...] = dv
    x0_ref[...] = _tau_bf16(dv * w_ref[...])


def _layer_body(s_ref, dinv_ref, acc_ref, x_ref, acc_out_ref):
    dv = dinv_ref[...]
    t = dv * s_ref[...]
    x_ref[...] = _tau_bf16(dv * t)
    acc_out_ref[...] = acc_ref[...] + t


def _final_body(s_ref, dinv_ref, acc_ref, mean_ref):
    t = dinv_ref[...] * s_ref[...]
    mean_ref[...] = (acc_ref[...] + t) * 0.25


_TC_ROWS = N * D // 128
_TC_BLOCK = 1000
_tc_spec = pl.BlockSpec((_TC_BLOCK, 128), lambda i: (i, 0))
_tc_grid = (_TC_ROWS // _TC_BLOCK,)
_nd_f32 = jax.ShapeDtypeStruct((_TC_ROWS, 128), jnp.float32)
_nd_bf16 = jax.ShapeDtypeStruct((_TC_ROWS, 128), jnp.bfloat16)

_prep = pl.pallas_call(
    _prep_body, grid=_tc_grid,
    in_specs=[_tc_spec, _tc_spec],
    out_specs=[_tc_spec, _tc_spec],
    out_shape=[_nd_f32, _nd_bf16],
)
_layer = pl.pallas_call(
    _layer_body, grid=_tc_grid,
    in_specs=[_tc_spec, _tc_spec, _tc_spec],
    out_specs=[_tc_spec, _tc_spec],
    out_shape=[_nd_bf16, _nd_f32],
)
_final = pl.pallas_call(
    _final_body, grid=_tc_grid,
    in_specs=[_tc_spec, _tc_spec, _tc_spec],
    out_specs=_tc_spec,
    out_shape=_nd_f32,
)


def kernel(W, edge_row, edge_col, edge_val, user_idxs, pos_item_idxs,
           neg_item_idxs):
    del edge_val
    row = edge_row.astype(jnp.int32)
    col = edge_col.astype(jnp.int32)
    row_local = jnp.where(row >= NU, row - NU, row)

    zpad = jnp.zeros((PADH,), jnp.int32)
    spad = jnp.full((PADH,), SENT, jnp.int32)
    col_p = jnp.concatenate([col[:EH], zpad, col[EH:], zpad])
    row_p = jnp.concatenate([row_local[:EH], spad, row_local[EH:], spad])
    row_p = row_p.reshape(NBODIES, BODY_CHUNKS * 2, 128)

    ones = jnp.ones((128, D), jnp.float32)
    zeros = jnp.zeros((SLICE, D), jnp.float32)

    def to_v(a):
        return a.reshape(_TC_ROWS, 128)

    def from_v(a):
        return a.reshape(N, D)

    degrep = _degrees(row_p, ones, zeros)
    dinv, x_v = _prep(to_v(degrep), to_v(W))
    x = from_v(x_v)

    acc = to_v(W)
    for layer in range(3):
        s = _spmm(x, col_p, row_p, zeros)
        if layer < 2:
            x_v, acc = _layer(to_v(s), dinv, acc)
            x = from_v(x_v)
        else:
            mean = from_v(_final(to_v(s), dinv, acc))

    u32 = user_idxs.astype(jnp.int32)
    p32 = pos_item_idxs.astype(jnp.int32)
    n32 = neg_item_idxs.astype(jnp.int32)
    idx_all = jnp.concatenate([u32, p32 + NU, n32 + NU, u32, p32, n32])

    return _triplet_gather(mean, W, idx_all)


# speedup vs baseline: 1.0679x; 1.0679x over previous
import functools

import jax
import jax.numpy as jnp
from jax import lax
from jax.experimental import pallas as pl
from jax.experimental.pallas import tpu as pltpu
from jax.experimental.pallas import tpu_sc as plsc

NU = 50000
N = 2 * NU
D = 32
EH = 800000
NC, NS = 2, 16
CHUNK = 256
BODY_CHUNKS = 5
BODY_EDGES = BODY_CHUNKS * CHUNK
NBODY = 40
KSUB = NBODY * BODY_EDGES
EH_PAD = NS * KSUB
PADH = EH_PAD - EH
NBODIES = NC * NS * NBODY
SENT = NU
ACC_ROWS = NU + 8
SLICE = NU // NS
BATCH = 4096
GB = BATCH // (NC * NS)

_mesh = plsc.VectorSubcoreMesh(core_axis_name="c", subcore_axis_name="s")
_sc_params = pltpu.CompilerParams(use_tc_tiling_on_sc=False)
_sc_params_nl = pltpu.CompilerParams(use_tc_tiling_on_sc=False,
                                     needs_layout_passes=False)


@functools.partial(
    pl.kernel,
    out_type=jax.ShapeDtypeStruct((N, D), jnp.float32),
    mesh=_mesh,
    compiler_params=_sc_params_nl,
    scratch_types=[
        pltpu.VMEM_SHARED((ACC_ROWS, D), jnp.float32),
        pltpu.VMEM((BODY_EDGES,), jnp.int32),
        pltpu.VMEM((BODY_EDGES,), jnp.int32),
        pltpu.VMEM((BODY_CHUNKS * 2, 128), jnp.int32),
        pltpu.VMEM((BODY_CHUNKS * 2, 128), jnp.int32),
        pltpu.VMEM((CHUNK, D), jnp.float32),
        pltpu.VMEM((CHUNK, D), jnp.float32),
        pltpu.VMEM((CHUNK, D), jnp.bfloat16),
        pltpu.VMEM((CHUNK, D), jnp.bfloat16),
        pltpu.SemaphoreType.DMA,
        pltpu.SemaphoreType.DMA,
        pltpu.SemaphoreType.DMA,
        pltpu.SemaphoreType.DMA,
        pltpu.SemaphoreType.DMA,
        pltpu.SemaphoreType.DMA,
    ],
)
def _spmm(x_hbm, col_hbm, row_hbm, zero_hbm, out_hbm,
          acc_sh, colv0, colv1, rowv0, rowv1, rows0, rows1, rowsb0, rowsb1,
          sI0, sI1, sG0, sG1, sS0, sS1):
    c = lax.axis_index("c")
    s = lax.axis_index("s")
    pltpu.sync_copy(zero_hbm, acc_sh.at[pl.ds(s * SLICE, SLICE)])
    plsc.subcore_barrier()
    ebase = c * EH_PAD + s * KSUB
    bbase = (c * NS + s) * NBODY
    colv, rowv = (colv0, colv1), (rowv0, rowv1)
    rows, sI = (rows0, rows1), (sI0, sI1)
    rowsb = (rowsb0, rowsb1)
    sG, sS = (sG0, sG1), (sS0, sS1)

    def idx_descs(b, buf):
        return (
            pltpu.make_async_copy(
                col_hbm.at[pl.ds(ebase + b * BODY_EDGES, BODY_EDGES)],
                colv[buf], sI[buf]),
            pltpu.make_async_copy(row_hbm.at[bbase + b], rowv[buf], sI[buf]),
        )

    for d in idx_descs(0, 0):
        d.start()

    @pl.loop(0, NBODY // 2)
    def _(t):
        for bb in range(2):
            P, Q = bb, 1 - bb
            b = 2 * t + bb
            for d in idx_descs(b, P):
                d.wait()
            bn = jnp.minimum(b + 1, NBODY - 1)
            for d in idx_descs(bn, Q):
                d.start()

            def gather(i, p):
                return pltpu.async_copy(
                    x_hbm.at[colv[P].at[pl.ds(i * CHUNK, CHUNK)]],
                    rowsb[p], sG[p])

            gd = [gather(0, 0), None]
            pend = [[], []]
            for i in range(BODY_CHUNKS):
                p = i % 2
                q = 1 - p
                gd[p].wait()
                if i < BODY_CHUNKS - 1:
                    gd[q] = gather(i + 1, q)
                for d in pend[p]:
                    d.wait()
                pend[p] = []

                @plsc.parallel_loop(0, CHUNK, step=1, unroll=8)
                def _(r):
                    ab = rowsb[p][r, :]
                    lo, hi = plsc.unpack(
                        ab, format=plsc.PackFormat.INTERLEAVED)
                    rows[p][r, pl.ds(0, 16)] = lo
                    rows[p][r, pl.ds(16, 16)] = hi

                for g in range(2):
                    pend[p].append(pltpu.async_copy(
                        rows[p].at[pl.ds(g * 128, 128)],
                        acc_sh.at[rowv[P].at[i * 2 + g]], sS[p], add=True))
            for p in range(2):
                for d in pend[p]:
                    d.wait()

    for d in idx_descs(NBODY - 1, 0):
        d.wait()
    plsc.subcore_barrier()
    pltpu.sync_copy(acc_sh.at[pl.ds(s * SLICE, SLICE)],
                    out_hbm.at[pl.ds(c * NU + s * SLICE, SLICE)])


@functools.partial(
    pl.kernel,
    out_type=jax.ShapeDtypeStruct((N, D), jnp.float32),
    mesh=_mesh,
    compiler_params=_sc_params,
    scratch_types=[
        pltpu.VMEM_SHARED((ACC_ROWS, D), jnp.float32),
        pltpu.VMEM((BODY_CHUNKS * 2, 128), jnp.int32),
        pltpu.VMEM((BODY_CHUNKS * 2, 128), jnp.int32),
        pltpu.VMEM((128, D), jnp.float32),
        pltpu.SemaphoreType.DMA,
        pltpu.SemaphoreType.DMA,
        pltpu.SemaphoreType.DMA,
    ],
)
def _degrees(row_hbm, ones_hbm, zero_hbm, out_hbm,
             acc_sh, rowv0, rowv1, onesv, sI0, sI1, sS):
    c = lax.axis_index("c")
    s = lax.axis_index("s")
    pltpu.sync_copy(zero_hbm, acc_sh.at[pl.ds(s * SLICE, SLICE)])
    pltpu.sync_copy(ones_hbm, onesv)
    plsc.subcore_barrier()
    bbase = (c * NS + s) * NBODY
    rowv, sI = (rowv0, rowv1), (sI0, sI1)

    def idx_desc(b, buf):
        return pltpu.make_async_copy(row_hbm.at[bbase + b], rowv[buf], sI[buf])

    idx_desc(0, 0).start()

    @pl.loop(0, NBODY // 2)
    def _(t):
        for bb in range(2):
            P, Q = bb, 1 - bb
            b = 2 * t + bb
            idx_desc(b, P).wait()
            bn = jnp.minimum(b + 1, NBODY - 1)
            idx_desc(bn, Q).start()
            pend = []
            for g in range(BODY_CHUNKS * 2):
                pend.append(pltpu.async_copy(
                    onesv, acc_sh.at[rowv[P].at[g]], sS, add=True))
            for d in pend:
                d.wait()

    idx_desc(NBODY - 1, 0).wait()
    plsc.subcore_barrier()
    pltpu.sync_copy(acc_sh.at[pl.ds(s * SLICE, SLICE)],
                    out_hbm.at[pl.ds(c * NU + s * SLICE, SLICE)])


@functools.partial(
    pl.kernel,
    out_type=tuple(jax.ShapeDtypeStruct((BATCH, D), jnp.float32)
                   for _ in range(6)),
    mesh=_mesh,
    compiler_params=_sc_params,
    scratch_types=[
        pltpu.VMEM((GB,), jnp.int32),
        pltpu.VMEM((GB, D), jnp.float32),
        pltpu.SemaphoreType.DMA,
    ],
)
def _triplet_gather(mean_hbm, w_hbm, idx_hbm,
                    o0, o1, o2, o3, o4, o5, idxv, rowsv, sem):
    wid = lax.axis_index("s") * NC + lax.axis_index("c")
    for t, out in enumerate((o0, o1, o2, o3, o4, o5)):
        tab = mean_hbm if t < 3 else w_hbm
        pltpu.sync_copy(idx_hbm.at[pl.ds(t * BATCH + wid * GB, GB)], idxv)
        pltpu.async_copy(tab.at[idxv], rowsv, sem).wait()
        pltpu.sync_copy(rowsv, out.at[pl.ds(wid * GB, GB)])


def _tau_bf16(x):
    b = x.shape[0]
    return (x.reshape(b, 4, 2, 16).transpose(0, 1, 3, 2)
            .reshape(b, 128).astype(jnp.bfloat16))


def _prep_body(deg_ref, w_ref, dinv_ref, x0_ref):
    d = deg_ref[...]
    dv = jnp.where(d > 0.0, lax.rsqrt(jnp.maximum(d, 1.0)), 0.0)
    dinv_ref[...] = dv
    x0_ref[...] = _tau_bf16(dv * w_ref[...])


def _layer_body(s_ref, dinv_ref, acc_ref, x_ref, acc_out_ref):
    dv = dinv_ref[...]
    t = dv * s_ref[...]
    x_ref[...] = _tau_bf16(dv * t)
    acc_out_ref[...] = acc_ref[...] + t


def _final_body(s_ref, dinv_ref, acc_ref, mean_ref):
    t = dinv_ref[...] * s_ref[...]
    mean_ref[...] = (acc_ref[...] + t) * 0.25


_TC_ROWS = N * D // 128
_TC_BLOCK = 1000
_tc_spec = pl.BlockSpec((_TC_BLOCK, 128), lambda i: (i, 0))
_tc_grid = (_TC_ROWS // _TC_BLOCK,)
_nd_f32 = jax.ShapeDtypeStruct((_TC_ROWS, 128), jnp.float32)
_nd_bf16 = jax.ShapeDtypeStruct((_TC_ROWS, 128), jnp.bfloat16)

_prep = pl.pallas_call(
    _prep_body, grid=_tc_grid,
    in_specs=[_tc_spec, _tc_spec],
    out_specs=[_tc_spec, _tc_spec],
    out_shape=[_nd_f32, _nd_bf16],
)
_layer = pl.pallas_call(
    _layer_body, grid=_tc_grid,
    in_specs=[_tc_spec, _tc_spec, _tc_spec],
    out_specs=[_tc_spec, _tc_spec],
    out_shape=[_nd_bf16, _nd_f32],
)
_final = pl.pallas_call(
    _final_body, grid=_tc_grid,
    in_specs=[_tc_spec, _tc_spec, _tc_spec],
    out_specs=_tc_spec,
    out_shape=_nd_f32,
)


def kernel(W, edge_row, edge_col, edge_val, user_idxs, pos_item_idxs,
           neg_item_idxs):
    del edge_val
    row = edge_row.astype(jnp.int32)
    col = edge_col.astype(jnp.int32)
    row_local = jnp.where(row >= NU, row - NU, row)

    zpad = jnp.zeros((PADH,), jnp.int32)
    spad = jnp.full((PADH,), SENT, jnp.int32)
    col_p = jnp.concatenate([col[:EH], zpad, col[EH:], zpad])
    row_p = jnp.concatenate([row_local[:EH], spad, row_local[EH:], spad])
    row_p = row_p.reshape(NBODIES, BODY_CHUNKS * 2, 128)

    ones = jnp.ones((128, D), jnp.float32)
    zeros = jnp.zeros((SLICE, D), jnp.float32)

    def to_v(a):
        return a.reshape(_TC_ROWS, 128)

    def from_v(a):
        return a.reshape(N, D)

    degrep = _degrees(row_p, ones, zeros)
    dinv, x_v = _prep(to_v(degrep), to_v(W))
    x = from_v(x_v)

    acc = to_v(W)
    for layer in range(3):
        s = _spmm(x, col_p, row_p, zeros)
        if layer < 2:
            x_v, acc = _layer(to_v(s), dinv, acc)
            x = from_v(x_v)
        else:
            mean = from_v(_final(to_v(s), dinv, acc))

    u32 = user_idxs.astype(jnp.int32)
    p32 = pos_item_idxs.astype(jnp.int32)
    n32 = neg_item_idxs.astype(jnp.int32)
    idx_all = jnp.concatenate([u32, p32 + NU, n32 + NU, u32, p32, n32])

    return _triplet_gather(mean, W, idx_all)


# speedup vs baseline: 1.5115x; 1.4154x over previous
import functools

import jax
import jax.numpy as jnp
from jax import lax
from jax.experimental import pallas as pl
from jax.experimental.pallas import tpu as pltpu
from jax.experimental.pallas import tpu_sc as plsc

NU = 50000
N = 2 * NU
D = 32
EH = 800000
NC, NS = 2, 16
CHUNK = 256
BODY_CHUNKS = 5
BODY_EDGES = BODY_CHUNKS * CHUNK
NBODY = 40
KSUB = NBODY * BODY_EDGES
EH_PAD = NS * KSUB
PADH = EH_PAD - EH
NBODIES = NC * NS * NBODY
SENT = NU
ACC_ROWS = NU + 8
SLICE = NU // NS
BATCH = 4096
GB = BATCH // (NC * NS)

_mesh = plsc.VectorSubcoreMesh(core_axis_name="c", subcore_axis_name="s")
_sc_params = pltpu.CompilerParams(use_tc_tiling_on_sc=False)


@functools.partial(
    pl.kernel,
    out_type=jax.ShapeDtypeStruct((N, D), jnp.float32),
    mesh=_mesh,
    compiler_params=_sc_params,
    scratch_types=[
        pltpu.VMEM_SHARED((ACC_ROWS, D), jnp.float32),
        pltpu.VMEM((BODY_EDGES,), jnp.int32),
        pltpu.VMEM((BODY_EDGES,), jnp.int32),
        pltpu.VMEM((BODY_CHUNKS * 2, 128), jnp.int32),
        pltpu.VMEM((BODY_CHUNKS * 2, 128), jnp.int32),
        pltpu.VMEM((CHUNK, D), jnp.float32),
        pltpu.VMEM((CHUNK, D), jnp.float32),
        pltpu.VMEM((CHUNK, D), jnp.float32),
        pltpu.SemaphoreType.DMA,
        pltpu.SemaphoreType.DMA,
        pltpu.SemaphoreType.DMA,
        pltpu.SemaphoreType.DMA,
        pltpu.SemaphoreType.DMA,
        pltpu.SemaphoreType.DMA,
        pltpu.SemaphoreType.DMA,
        pltpu.SemaphoreType.DMA,
    ],
)
def _spmm(x_hbm, col_hbm, row_hbm, zero_hbm, out_hbm,
          acc_sh, colv0, colv1, rowv0, rowv1, rows0, rows1, rows2,
          sI0, sI1, sG0, sG1, sG2, sS0, sS1, sS2):
    c = lax.axis_index("c")
    s = lax.axis_index("s")
    pltpu.sync_copy(zero_hbm, acc_sh.at[pl.ds(s * SLICE, SLICE)])
    plsc.subcore_barrier()
    ebase = c * EH_PAD + s * KSUB
    bbase = (c * NS + s) * NBODY
    colv, rowv = (colv0, colv1), (rowv0, rowv1)
    rows, sI = (rows0, rows1, rows2), (sI0, sI1)
    sG, sS = (sG0, sG1, sG2), (sS0, sS1, sS2)

    def idx_descs(b, buf):
        return (
            pltpu.make_async_copy(
                col_hbm.at[pl.ds(ebase + b * BODY_EDGES, BODY_EDGES)],
                colv[buf], sI[buf]),
            pltpu.make_async_copy(row_hbm.at[bbase + b], rowv[buf], sI[buf]),
        )

    for d in idx_descs(0, 0):
        d.start()

    @pl.loop(0, NBODY // 2)
    def _(t):
        for bb in range(2):
            P, Q = bb, 1 - bb
            b = 2 * t + bb
            for d in idx_descs(b, P):
                d.wait()
            bn = jnp.minimum(b + 1, NBODY - 1)
            for d in idx_descs(bn, Q):
                d.start()

            def gather(i, p):
                return pltpu.async_copy(
                    x_hbm.at[colv[P].at[pl.ds(i * CHUNK, CHUNK)]],
                    rows[p], sG[p])

            gd = [gather(0, 0), gather(1, 1), None]
            pend = [[], [], []]
            for i in range(BODY_CHUNKS):
                p = i % 3
                gd[p].wait()
                if i + 2 < BODY_CHUNKS:
                    nb = (i + 2) % 3
                    for d in pend[nb]:
                        d.wait()
                    pend[nb] = []
                    gd[nb] = gather(i + 2, nb)
                for g in range(2):
                    pend[p].append(pltpu.async_copy(
                        rows[p].at[pl.ds(g * 128, 128)],
                        acc_sh.at[rowv[P].at[i * 2 + g]], sS[p], add=True))
            for p in range(3):
                for d in pend[p]:
                    d.wait()

    for d in idx_descs(NBODY - 1, 0):
        d.wait()
    plsc.subcore_barrier()
    pltpu.sync_copy(acc_sh.at[pl.ds(s * SLICE, SLICE)],
                    out_hbm.at[pl.ds(c * NU + s * SLICE, SLICE)])


@functools.partial(
    pl.kernel,
    out_type=jax.ShapeDtypeStruct((N, D), jnp.float32),
    mesh=_mesh,
    compiler_params=_sc_params,
    scratch_types=[
        pltpu.VMEM_SHARED((ACC_ROWS, D), jnp.float32),
        pltpu.VMEM((BODY_CHUNKS * 2, 128), jnp.int32),
        pltpu.VMEM((BODY_CHUNKS * 2, 128), jnp.int32),
        pltpu.VMEM((128, D), jnp.float32),
        pltpu.SemaphoreType.DMA,
        pltpu.SemaphoreType.DMA,
        pltpu.SemaphoreType.DMA,
    ],
)
def _degrees(row_hbm, ones_hbm, zero_hbm, out_hbm,
             acc_sh, rowv0, rowv1, onesv, sI0, sI1, sS):
    c = lax.axis_index("c")
    s = lax.axis_index("s")
    pltpu.sync_copy(zero_hbm, acc_sh.at[pl.ds(s * SLICE, SLICE)])
    pltpu.sync_copy(ones_hbm, onesv)
    plsc.subcore_barrier()
    bbase = (c * NS + s) * NBODY
    rowv, sI = (rowv0, rowv1), (sI0, sI1)

    def idx_desc(b, buf):
        return pltpu.make_async_copy(row_hbm.at[bbase + b], rowv[buf], sI[buf])

    idx_desc(0, 0).start()

    @pl.loop(0, NBODY // 2)
    def _(t):
        for bb in range(2):
            P, Q = bb, 1 - bb
            b = 2 * t + bb
            idx_desc(b, P).wait()
            bn = jnp.minimum(b + 1, NBODY - 1)
            idx_desc(bn, Q).start()
            pend = []
            for g in range(BODY_CHUNKS * 2):
                pend.append(pltpu.async_copy(
                    onesv, acc_sh.at[rowv[P].at[g]], sS, add=True))
            for d in pend:
                d.wait()

    idx_desc(NBODY - 1, 0).wait()
    plsc.subcore_barrier()
    pltpu.sync_copy(acc_sh.at[pl.ds(s * SLICE, SLICE)],
                    out_hbm.at[pl.ds(c * NU + s * SLICE, SLICE)])


@functools.partial(
    pl.kernel,
    out_type=tuple(jax.ShapeDtypeStruct((BATCH, D), jnp.float32)
                   for _ in range(6)),
    mesh=_mesh,
    compiler_params=_sc_params,
    scratch_types=[
        pltpu.VMEM((GB,), jnp.int32),
        pltpu.VMEM((GB, D), jnp.float32),
        pltpu.SemaphoreType.DMA,
    ],
)
def _triplet_gather(mean_hbm, w_hbm, idx_hbm,
                    o0, o1, o2, o3, o4, o5, idxv, rowsv, sem):
    wid = lax.axis_index("s") * NC + lax.axis_index("c")
    for t, out in enumerate((o0, o1, o2, o3, o4, o5)):
        tab = mean_hbm if t < 3 else w_hbm
        pltpu.sync_copy(idx_hbm.at[pl.ds(t * BATCH + wid * GB, GB)], idxv)
        pltpu.async_copy(tab.at[idxv], rowsv, sem).wait()
        pltpu.sync_copy(rowsv, out.at[pl.ds(wid * GB, GB)])


def _prep_body(deg_ref, w_ref, dinv_ref, x0_ref):
    d = deg_ref[...]
    dv = jnp.where(d > 0.0, lax.rsqrt(jnp.maximum(d, 1.0)), 0.0)
    dinv_ref[...] = dv
    x0_ref[...] = dv * w_ref[...]


def _layer_body(s_ref, dinv_ref, acc_ref, x_ref, acc_out_ref):
    dv = dinv_ref[...]
    t = dv * s_ref[...]
    x_ref[...] = dv * t
    acc_out_ref[...] = acc_ref[...] + t


def _final_body(s_ref, dinv_ref, acc_ref, mean_ref):
    t = dinv_ref[...] * s_ref[...]
    mean_ref[...] = (acc_ref[...] + t) * 0.25


_TC_ROWS = N * D // 128
_TC_BLOCK = 1000
_tc_spec = pl.BlockSpec((_TC_BLOCK, 128), lambda i: (i, 0))
_tc_grid = (_TC_ROWS // _TC_BLOCK,)
_nd_f32 = jax.ShapeDtypeStruct((_TC_ROWS, 128), jnp.float32)

_prep = pl.pallas_call(
    _prep_body, grid=_tc_grid,
    in_specs=[_tc_spec, _tc_spec],
    out_specs=[_tc_spec, _tc_spec],
    out_shape=[_nd_f32, _nd_f32],
)
_layer = pl.pallas_call(
    _layer_body, grid=_tc_grid,
    in_specs=[_tc_spec, _tc_spec, _tc_spec],
    out_specs=[_tc_spec, _tc_spec],
    out_shape=[_nd_f32, _nd_f32],
)
_final = pl.pallas_call(
    _final_body, grid=_tc_grid,
    in_specs=[_tc_spec, _tc_spec, _tc_spec],
    out_specs=_tc_spec,
    out_shape=_nd_f32,
)


def kernel(W, edge_row, edge_col, edge_val, user_idxs, pos_item_idxs,
           neg_item_idxs):
    del edge_val
    row = edge_row.astype(jnp.int32)
    col = edge_col.astype(jnp.int32)
    row_local = jnp.where(row >= NU, row - NU, row)

    zpad = jnp.zeros((PADH,), jnp.int32)
    spad = jnp.full((PADH,), SENT, jnp.int32)
    col_p = jnp.concatenate([col[:EH], zpad, col[EH:], zpad])
    row_p = jnp.concatenate([row_local[:EH], spad, row_local[EH:], spad])
    row_p = row_p.reshape(NBODIES, BODY_CHUNKS * 2, 128)

    ones = jnp.ones((128, D), jnp.float32)
    zeros = jnp.zeros((SLICE, D), jnp.float32)

    def to_v(a):
        return a.reshape(_TC_ROWS, 128)

    def from_v(a):
        return a.reshape(N, D)

    degrep = _degrees(row_p, ones, zeros)
    dinv, x_v = _prep(to_v(degrep), to_v(W))
    x = from_v(x_v)

    acc = to_v(W)
    for layer in range(3):
        s = _spmm(x, col_p, row_p, zeros)
        if layer < 2:
            x_v, acc = _layer(to_v(s), dinv, acc)
            x = from_v(x_v)
        else:
            mean = from_v(_final(to_v(s), dinv, acc))

    u32 = user_idxs.astype(jnp.int32)
    p32 = pos_item_idxs.astype(jnp.int32)
    n32 = neg_item_idxs.astype(jnp.int32)
    idx_all = jnp.concatenate([u32, p32 + NU, n32 + NU, u32, p32, n32])

    return _triplet_gather(mean, W, idx_all)


# speedup vs baseline: 1.5296x; 1.0120x over previous
import functools

import jax
import jax.numpy as jnp
from jax import lax
from jax.experimental import pallas as pl
from jax.experimental.pallas import tpu as pltpu
from jax.experimental.pallas import tpu_sc as plsc

NU = 50000
N = 2 * NU
D = 32
EH = 800000
NC, NS = 2, 16
CHUNK = 256
BODY_CHUNKS = 5
BODY_EDGES = BODY_CHUNKS * CHUNK
NBODY = 40
KSUB = NBODY * BODY_EDGES
EH_PAD = NS * KSUB
PADH = EH_PAD - EH
NBODIES = NC * NS * NBODY
SENT = NU
ACC_ROWS = NU + 8
SLICE = NU // NS
BATCH = 4096
GB = BATCH // (NC * NS)

def _zero_slice(zbuf, acc_sh, s, sem):
    zv = jnp.zeros((16,), jnp.float32)

    @plsc.parallel_loop(0, 256, step=1, unroll=8)
    def _(r):
        zbuf[r, pl.ds(0, 16)] = zv
        zbuf[r, pl.ds(16, 16)] = zv

    pend = [pltpu.async_copy(zbuf, acc_sh.at[pl.ds(s * SLICE + j * 256, 256)],
                             sem) for j in range(SLICE // 256)]
    pend.append(pltpu.async_copy(
        zbuf.at[pl.ds(0, SLICE % 256)],
        acc_sh.at[pl.ds(s * SLICE + (SLICE // 256) * 256, SLICE % 256)], sem))
    for d in pend:
        d.wait()


_mesh = plsc.VectorSubcoreMesh(core_axis_name="c", subcore_axis_name="s")
_sc_params = pltpu.CompilerParams(use_tc_tiling_on_sc=False)


@functools.partial(
    pl.kernel,
    out_type=jax.ShapeDtypeStruct((N, D), jnp.float32),
    mesh=_mesh,
    compiler_params=_sc_params,
    scratch_types=[
        pltpu.VMEM_SHARED((ACC_ROWS, D), jnp.float32),
        pltpu.VMEM((BODY_EDGES,), jnp.int32),
        pltpu.VMEM((BODY_EDGES,), jnp.int32),
        pltpu.VMEM((BODY_CHUNKS * 2, 128), jnp.int32),
        pltpu.VMEM((BODY_CHUNKS * 2, 128), jnp.int32),
        pltpu.VMEM((CHUNK, D), jnp.float32),
        pltpu.VMEM((CHUNK, D), jnp.float32),
        pltpu.VMEM((CHUNK, D), jnp.float32),
        pltpu.SemaphoreType.DMA,
        pltpu.SemaphoreType.DMA,
        pltpu.SemaphoreType.DMA,
        pltpu.SemaphoreType.DMA,
        pltpu.SemaphoreType.DMA,
        pltpu.SemaphoreType.DMA,
        pltpu.SemaphoreType.DMA,
        pltpu.SemaphoreType.DMA,
    ],
)
def _spmm(x_hbm, col_hbm, row_hbm, out_hbm,
          acc_sh, colv0, colv1, rowv0, rowv1, rows0, rows1, rows2,
          sI0, sI1, sG0, sG1, sG2, sS0, sS1, sS2):
    c = lax.axis_index("c")
    s = lax.axis_index("s")
    _zero_slice(rows0, acc_sh, s, sS0)
    plsc.subcore_barrier()
    ebase = c * EH_PAD + s * KSUB
    bbase = (c * NS + s) * NBODY
    colv, rowv = (colv0, colv1), (rowv0, rowv1)
    rows, sI = (rows0, rows1, rows2), (sI0, sI1)
    sG, sS = (sG0, sG1, sG2), (sS0, sS1, sS2)

    def idx_descs(b, buf):
        return (
            pltpu.make_async_copy(
                col_hbm.at[pl.ds(ebase + b * BODY_EDGES, BODY_EDGES)],
                colv[buf], sI[buf]),
            pltpu.make_async_copy(row_hbm.at[bbase + b], rowv[buf], sI[buf]),
        )

    for d in idx_descs(0, 0):
        d.start()

    @pl.loop(0, NBODY // 2)
    def _(t):
        for bb in range(2):
            P, Q = bb, 1 - bb
            b = 2 * t + bb
            for d in idx_descs(b, P):
                d.wait()
            bn = jnp.minimum(b + 1, NBODY - 1)
            for d in idx_descs(bn, Q):
                d.start()

            def gather(i, p):
                return pltpu.async_copy(
                    x_hbm.at[colv[P].at[pl.ds(i * CHUNK, CHUNK)]],
                    rows[p], sG[p])

            gd = [gather(0, 0), gather(1, 1), None]
            pend = [[], [], []]
            for i in range(BODY_CHUNKS):
                p = i % 3
                gd[p].wait()
                if i + 2 < BODY_CHUNKS:
                    nb = (i + 2) % 3
                    for d in pend[nb]:
                        d.wait()
                    pend[nb] = []
                    gd[nb] = gather(i + 2, nb)
                for g in range(2):
                    pend[p].append(pltpu.async_copy(
                        rows[p].at[pl.ds(g * 128, 128)],
                        acc_sh.at[rowv[P].at[i * 2 + g]], sS[p], add=True))
            for p in range(3):
                for d in pend[p]:
                    d.wait()

    for d in idx_descs(NBODY - 1, 0):
        d.wait()
    plsc.subcore_barrier()
    pltpu.sync_copy(acc_sh.at[pl.ds(s * SLICE, SLICE)],
                    out_hbm.at[pl.ds(c * NU + s * SLICE, SLICE)])


@functools.partial(
    pl.kernel,
    out_type=jax.ShapeDtypeStruct((N, D), jnp.float32),
    mesh=_mesh,
    compiler_params=_sc_params,
    scratch_types=[
        pltpu.VMEM_SHARED((ACC_ROWS, D), jnp.float32),
        pltpu.VMEM((BODY_CHUNKS * 2, 128), jnp.int32),
        pltpu.VMEM((BODY_CHUNKS * 2, 128), jnp.int32),
        pltpu.VMEM((128, D), jnp.float32),
        pltpu.VMEM((256, D), jnp.float32),
        pltpu.SemaphoreType.DMA,
        pltpu.SemaphoreType.DMA,
        pltpu.SemaphoreType.DMA,
    ],
)
def _degrees(row_hbm, ones_hbm, out_hbm,
             acc_sh, rowv0, rowv1, onesv, zbuf, sI0, sI1, sS):
    c = lax.axis_index("c")
    s = lax.axis_index("s")
    _zero_slice(zbuf, acc_sh, s, sS)
    pltpu.sync_copy(ones_hbm, onesv)
    plsc.subcore_barrier()
    bbase = (c * NS + s) * NBODY
    rowv, sI = (rowv0, rowv1), (sI0, sI1)

    def idx_desc(b, buf):
        return pltpu.make_async_copy(row_hbm.at[bbase + b], rowv[buf], sI[buf])

    idx_desc(0, 0).start()

    @pl.loop(0, NBODY // 2)
    def _(t):
        for bb in range(2):
            P, Q = bb, 1 - bb
            b = 2 * t + bb
            idx_desc(b, P).wait()
            bn = jnp.minimum(b + 1, NBODY - 1)
            idx_desc(bn, Q).start()
            pend = []
            for g in range(BODY_CHUNKS * 2):
                pend.append(pltpu.async_copy(
                    onesv, acc_sh.at[rowv[P].at[g]], sS, add=True))
            for d in pend:
                d.wait()

    idx_desc(NBODY - 1, 0).wait()
    plsc.subcore_barrier()
    pltpu.sync_copy(acc_sh.at[pl.ds(s * SLICE, SLICE)],
                    out_hbm.at[pl.ds(c * NU + s * SLICE, SLICE)])


@functools.partial(
    pl.kernel,
    out_type=tuple(jax.ShapeDtypeStruct((BATCH, D), jnp.float32)
                   for _ in range(6)),
    mesh=_mesh,
    compiler_params=_sc_params,
    scratch_types=[
        pltpu.VMEM((GB,), jnp.int32),
        pltpu.VMEM((GB, D), jnp.float32),
        pltpu.SemaphoreType.DMA,
    ],
)
def _triplet_gather(mean_hbm, w_hbm, idx_hbm,
                    o0, o1, o2, o3, o4, o5, idxv, rowsv, sem):
    wid = lax.axis_index("s") * NC + lax.axis_index("c")
    for t, out in enumerate((o0, o1, o2, o3, o4, o5)):
        tab = mean_hbm if t < 3 else w_hbm
        pltpu.sync_copy(idx_hbm.at[pl.ds(t * BATCH + wid * GB, GB)], idxv)
        pltpu.async_copy(tab.at[idxv], rowsv, sem).wait()
        pltpu.sync_copy(rowsv, out.at[pl.ds(wid * GB, GB)])


def _prep_body(deg_ref, w_ref, dinv_ref, x0_ref):
    d = deg_ref[...]
    dv = jnp.where(d > 0.0, lax.rsqrt(jnp.maximum(d, 1.0)), 0.0)
    dinv_ref[...] = dv
    x0_ref[...] = dv * w_ref[...]


def _layer_body(s_ref, dinv_ref, acc_ref, x_ref, acc_out_ref):
    dv = dinv_ref[...]
    t = dv * s_ref[...]
    x_ref[...] = dv * t
    acc_out_ref[...] = acc_ref[...] + t


def _final_body(s_ref, dinv_ref, acc_ref, mean_ref):
    t = dinv_ref[...] * s_ref[...]
    mean_ref[...] = (acc_ref[...] + t) * 0.25


_TC_ROWS = N * D // 128
_TC_BLOCK = 1000
_tc_spec = pl.BlockSpec((_TC_BLOCK, 128), lambda i: (i, 0))
_tc_grid = (_TC_ROWS // _TC_BLOCK,)
_nd_f32 = jax.ShapeDtypeStruct((_TC_ROWS, 128), jnp.float32)

_prep = pl.pallas_call(
    _prep_body, grid=_tc_grid,
    in_specs=[_tc_spec, _tc_spec],
    out_specs=[_tc_spec, _tc_spec],
    out_shape=[_nd_f32, _nd_f32],
)
_layer = pl.pallas_call(
    _layer_body, grid=_tc_grid,
    in_specs=[_tc_spec, _tc_spec, _tc_spec],
    out_specs=[_tc_spec, _tc_spec],
    out_shape=[_nd_f32, _nd_f32],
)
_final = pl.pallas_call(
    _final_body, grid=_tc_grid,
    in_specs=[_tc_spec, _tc_spec, _tc_spec],
    out_specs=_tc_spec,
    out_shape=_nd_f32,
)


def kernel(W, edge_row, edge_col, edge_val, user_idxs, pos_item_idxs,
           neg_item_idxs):
    del edge_val
    row = edge_row.astype(jnp.int32)
    col = edge_col.astype(jnp.int32)
    row_local = jnp.where(row >= NU, row - NU, row)

    zpad = jnp.zeros((PADH,), jnp.int32)
    spad = jnp.full((PADH,), SENT, jnp.int32)
    col_p = jnp.concatenate([col[:EH], zpad, col[EH:], zpad])
    row_p = jnp.concatenate([row_local[:EH], spad, row_local[EH:], spad])
    row_p = row_p.reshape(NBODIES, BODY_CHUNKS * 2, 128)

    ones = jnp.ones((128, D), jnp.float32)

    def to_v(a):
        return a.reshape(_TC_ROWS, 128)

    def from_v(a):
        return a.reshape(N, D)

    degrep = _degrees(row_p, ones)
    dinv, x_v = _prep(to_v(degrep), to_v(W))
    x = from_v(x_v)

    acc = to_v(W)
    for layer in range(3):
        s = _spmm(x, col_p, row_p)
        if layer < 2:
            x_v, acc = _layer(to_v(s), dinv, acc)
            x = from_v(x_v)
        else:
            mean = from_v(_final(to_v(s), dinv, acc))

    u32 = user_idxs.astype(jnp.int32)
    p32 = pos_item_idxs.astype(jnp.int32)
    n32 = neg_item_idxs.astype(jnp.int32)
    idx_all = jnp.concatenate([u32, p32 + NU, n32 + NU, u32, p32, n32])

    return _triplet_gather(mean, W, idx_all)
